# baseline (speedup 1.0000x reference)
import functools

import jax
import jax.numpy as jnp
from jax import lax
from jax.experimental import pallas as pl
from jax.experimental.pallas import tpu as pltpu
from jax.experimental.pallas import tpu_sc as plsc

N = 10000
D = 128
E = 160000
NP = 10240
EP = 163840
E2P = 327680
NC = 2
NS = 16
ROWS_PER_SUB = NP // NS
SEG_CHUNK = 128
SEG_ITERS = E2P // (NC * NS * SEG_CHUNK)
EDGE_CHUNK = 80
EDGE_ITERS = EP // (NC * NS * EDGE_CHUNK)

SEG_M0 = 40
SEG_M1 = 120
EDGE_M0 = 48
EDGE_M1 = 80

_f32 = jnp.float32


def _stats_body(x_ref, o_ref):
    i = pl.program_id(0)

    @pl.when(i == 0)
    def _init():
        o_ref[...] = jnp.zeros_like(o_ref)

    xb = x_ref[...]
    o_ref[0:1, :] += jnp.sum(xb, axis=0, keepdims=True)
    o_ref[1:2, :] += jnp.sum(xb * xb, axis=0, keepdims=True)


def _norm_mm_body(stats_ref, x_ref, ws_ref, wn_ref, us_ref, un_ref):
    tot = float(N * D)
    m = jnp.sum(stats_ref[0, :]) / tot
    ex2 = jnp.sum(stats_ref[1, :]) / tot
    s = jnp.sqrt(jnp.maximum(ex2 - m * m, 0.0))
    xn = (x_ref[...] - m) * (1.0 / (s + 1e-6))
    us_ref[...] = jnp.dot(xn, ws_ref[...], preferred_element_type=_f32)
    un_ref[...] = jnp.dot(xn, wn_ref[...], preferred_element_type=_f32)


def _layer_body(u_ref, p0_ref, p1_ref, ws_ref, wn_ref, vs_ref, vn_ref):
    h = jnp.maximum(u_ref[...] + p0_ref[0] + p1_ref[0], 0.0)
    vs_ref[...] = jnp.dot(h, ws_ref[...], preferred_element_type=_f32)
    vn_ref[...] = jnp.dot(h, wn_ref[...], preferred_element_type=_f32)


def _tables_body(v_ref, p0_ref, p1_ref, wa_ref, wb_ref, qs_ref, qd_ref):
    h = jnp.maximum(v_ref[...] + p0_ref[0] + p1_ref[0], 0.0)
    ha = jnp.dot(h, wa_ref[...], preferred_element_type=_f32)
    hb = jnp.dot(h, wb_ref[...], preferred_element_type=_f32)
    qs_ref[...] = jnp.concatenate([h, ha], axis=1)
    qd_ref[...] = jnp.concatenate([h, hb], axis=1)


def _epilogue_body(mr_ref, sr_ref, dt_ref, eps_ref, gt_ref,
                   mu_ref, std_ref, act_ref, side_ref):
    i = pl.program_id(0)

    @pl.when(i == 0)
    def _init():
        side_ref[0, 0] = 0.0

    mu = -0.5 + 0.5 * (jnp.tanh(mr_ref[...]) + 1.0)
    std = 0.1 + 0.45 * (jnp.tanh(sr_ref[...]) + 1.0)
    act = jax.nn.sigmoid(mu + std * eps_ref[...])
    mu_ref[...] = mu
    std_ref[...] = std
    act_ref[...] = act

    rows = lax.broadcasted_iota(jnp.int32, mr_ref.shape, 0) + i * mr_ref.shape[0]
    mask = rows < (E // D)
    sp = jax.nn.sigmoid(dt_ref[...])
    side_ref[0, 0] += jnp.sum(jnp.where(mask, (sp - gt_ref[...]) ** 2, 0.0))

    @pl.when(i == pl.num_programs(0) - 1)
    def _fin():
        side_ref[0, 0] = side_ref[0, 0] / float(E)


_sc_mesh = plsc.VectorSubcoreMesh(core_axis_name="c", subcore_axis_name="s")
_sc_params = pltpu.CompilerParams(needs_layout_passes=False)


@functools.partial(
    pl.kernel,
    out_type=jax.ShapeDtypeStruct((NC, NP, D), _f32),
    mesh=_sc_mesh,
    compiler_params=_sc_params,
    scratch_types=[
        pltpu.VMEM_SHARED((NP, D), _f32),
        pltpu.VMEM((2, 2, SEG_CHUNK), jnp.int32),
        pltpu.VMEM((2, SEG_CHUNK, D), _f32),
        pltpu.SemaphoreType.DMA,
        pltpu.SemaphoreType.DMA,
        pltpu.SemaphoreType.DMA,
        pltpu.SemaphoreType.DMA,
    ],
)
def _segsum(table, sd, zeros, out, acc, sdb, rows, si0, si1, sg0, sg1):
    c = lax.axis_index("c")
    s = lax.axis_index("s")
    pltpu.sync_copy(zeros.at[pl.ds(s * ROWS_PER_SUB, ROWS_PER_SUB)],
                    acc.at[pl.ds(s * ROWS_PER_SUB, ROWS_PER_SUB)])
    plsc.subcore_barrier()

    K = SEG_CHUNK
    nchunk = jnp.where(c == 0, SEG_M0, SEG_M1)
    base = jnp.where(c == 0, s * SEG_M0, NS * SEG_M0 + s * SEG_M1) * K

    def issue_idx(buf, off, sem):
        pltpu.async_copy(sd.at[:, pl.ds(off, K)], sdb.at[buf], sem)

    def wait_idx(buf, sem):
        pltpu.make_async_copy(sd.at[:, pl.ds(0, K)], sdb.at[buf], sem).wait()

    def issue_gather(buf, sem):
        pltpu.async_copy(table.at[sdb.at[buf, 0]], rows.at[buf], sem)

    def wait_gather(buf, sem):
        pltpu.make_async_copy(table.at[pl.ds(0, K)], rows.at[buf], sem).wait()

    def scatter(buf):
        pltpu.sync_copy(rows.at[buf], acc.at[sdb.at[buf, 1]], add=True)

    issue_idx(0, base, si0)
    wait_idx(0, si0)
    issue_gather(0, sg0)
    issue_idx(1, base + K, si1)

    npair = nchunk // 2

    def body(j, carry):
        more = j < npair - 1
        wait_gather(0, sg0)
        wait_idx(1, si1)
        issue_gather(1, sg1)
        scatter(0)

        @pl.when(more)
        def _():
            issue_idx(0, base + (2 * j + 2) * K, si0)

        wait_gather(1, sg1)

        @pl.when(more)
        def _():
            wait_idx(0, si0)
            issue_gather(0, sg0)

        scatter(1)

        @pl.when(more)
        def _():
            issue_idx(1, base + (2 * j + 3) * K, si1)

        return carry

    lax.fori_loop(0, npair, body, 0)
    plsc.subcore_barrier()

    for j in range(ROWS_PER_SUB // K):
        r0 = s * ROWS_PER_SUB + j * K
        pltpu.sync_copy(acc.at[pl.ds(r0, K)], rows.at[0])
        pltpu.sync_copy(rows.at[0], out.at[c, pl.ds(r0, K)])


@functools.partial(
    pl.kernel,
    out_type=(jax.ShapeDtypeStruct((EP,), _f32),
              jax.ShapeDtypeStruct((EP,), _f32),
              jax.ShapeDtypeStruct((EP,), _f32)),
    mesh=_sc_mesh,
    compiler_params=_sc_params,
    scratch_types=[
        pltpu.VMEM((2, EDGE_CHUNK, 2 * D), _f32),
        pltpu.VMEM((2, EDGE_CHUNK, 2 * D), _f32),
        pltpu.VMEM((2, EDGE_CHUNK), jnp.int32),
        pltpu.VMEM((2, EDGE_CHUNK), jnp.int32),
        pltpu.VMEM((2, EDGE_CHUNK), _f32),
        pltpu.VMEM((EDGE_CHUNK,), _f32),
        pltpu.VMEM((EDGE_CHUNK,), _f32),
        pltpu.VMEM((EDGE_CHUNK,), _f32),
        pltpu.VMEM((3, D), _f32),
        pltpu.SemaphoreType.DMA,
        pltpu.SemaphoreType.DMA,
        pltpu.SemaphoreType.DMA,
        pltpu.SemaphoreType.DMA,
    ],
)
def _edge_head(qs_t, qd_t, sidx, didx, ang, wcc, mu_o, sr_o, dt_o,
               qs_b, qd_b, sib, dib, anb, mub, srb, dtb, wcc_b,
               si0, si1, sq0, sq1):
    c = lax.axis_index("c")
    s = lax.axis_index("s")
    pltpu.sync_copy(wcc, wcc_b)
    lanes = lax.iota(jnp.int32, 16)
    wv = [wcc_b[0, pl.ds(fb * 16, 16)] for fb in range(8)]
    c0v = [wcc_b[1, pl.ds(fb * 16, 16)] for fb in range(8)]
    c1v = [wcc_b[2, pl.ds(fb * 16, 16)] for fb in range(8)]

    K = EDGE_CHUNK
    nchunk = jnp.where(c == 0, EDGE_M0, EDGE_M1)
    base = jnp.where(c == 0, s * EDGE_M0, NS * EDGE_M0 + s * EDGE_M1) * K

    def issue_idx(buf, off, sem):
        pltpu.async_copy(sidx.at[pl.ds(off, K)], sib.at[buf], sem)
        pltpu.async_copy(didx.at[pl.ds(off, K)], dib.at[buf], sem)
        pltpu.async_copy(ang.at[pl.ds(off, K)], anb.at[buf], sem)

    def wait_idx(buf, sem):
        pltpu.make_async_copy(sidx.at[pl.ds(0, K)], sib.at[buf], sem).wait()
        pltpu.make_async_copy(sidx.at[pl.ds(0, K)], dib.at[buf], sem).wait()
        pltpu.make_async_copy(ang.at[pl.ds(0, K)], anb.at[buf], sem).wait()

    def issue_gathers(buf, sem):
        pltpu.async_copy(qs_t.at[sib.at[buf]], qs_b.at[buf], sem)
        pltpu.async_copy(qd_t.at[dib.at[buf]], qd_b.at[buf], sem)

    def wait_gathers(buf, sem):
        pltpu.make_async_copy(qs_t.at[pl.ds(0, K)], qs_b.at[buf], sem).wait()
        pltpu.make_async_copy(qs_t.at[pl.ds(0, K)], qd_b.at[buf], sem).wait()

    def compute(buf, off):
        def group(g, carry2):
            g16 = g * 16
            ang_v = anb[buf, pl.ds(g16, 16)]
            mu_vec = jnp.zeros((16,), _f32)
            sr_vec = jnp.zeros((16,), _f32)
            dt_vec = jnp.zeros((16,), _f32)
            for lane in range(16):
                e = g16 + lane
                a_sc = ang_v[lane]
                mu_acc = jnp.zeros((16,), _f32)
                sr_acc = jnp.zeros((16,), _f32)
                dt_acc = jnp.zeros((16,), _f32)
                for fb in range(8):
                    hs = qs_b[buf, e, pl.ds(fb * 16, 16)]
                    hd = qd_b[buf, e, pl.ds(fb * 16, 16)]
                    ha = qs_b[buf, e, pl.ds(D + fb * 16, 16)]
                    hb = qd_b[buf, e, pl.ds(D + fb * 16, 16)]
                    rr = jnp.maximum(ha + hb + a_sc * wv[fb], 0.0)
                    mu_acc = mu_acc + rr * c0v[fb]
                    sr_acc = sr_acc + rr * c1v[fb]
                    dt_acc = dt_acc + hs * hd
                lm = lanes == lane
                mu_vec = jnp.where(lm, jnp.sum(mu_acc), mu_vec)
                sr_vec = jnp.where(lm, jnp.sum(sr_acc), sr_vec)
                dt_vec = jnp.where(lm, jnp.sum(dt_acc), dt_vec)
            mub[pl.ds(g16, 16)] = mu_vec
            srb[pl.ds(g16, 16)] = sr_vec
            dtb[pl.ds(g16, 16)] = dt_vec
            return carry2

        lax.fori_loop(0, K // 16, group, 0)
        pltpu.sync_copy(mub, mu_o.at[pl.ds(off, K)])
        pltpu.sync_copy(srb, sr_o.at[pl.ds(off, K)])
        pltpu.sync_copy(dtb, dt_o.at[pl.ds(off, K)])

    issue_idx(0, base, si0)
    wait_idx(0, si0)
    issue_gathers(0, sq0)
    issue_idx(1, base + K, si1)

    npair = nchunk // 2

    def body(j, carry):
        more = j < npair - 1
        off0 = base + (2 * j) * K
        wait_gathers(0, sq0)
        wait_idx(1, si1)
        issue_gathers(1, sq1)
        compute(0, off0)

        @pl.when(more)
        def _():
            issue_idx(0, base + (2 * j + 2) * K, si0)

        wait_gathers(1, sq1)

        @pl.when(more)
        def _():
            wait_idx(0, si0)
            issue_gathers(0, sq0)

        compute(1, off0 + K)

        @pl.when(more)
        def _():
            issue_idx(1, base + (2 * j + 3) * K, si1)

        return carry

    lax.fori_loop(0, npair, body, 0)


def kernel(node_embeddings, sp_feat, edge_ids, edge_angles, gt_edge_weights,
           Wself0, Wnbr0, Wself1, Wnbr1, We1, We2):
    x = jnp.concatenate([node_embeddings, sp_feat], axis=1).astype(_f32)

    stats = pl.pallas_call(
        _stats_body,
        grid=(10,),
        in_specs=[pl.BlockSpec((N // 10, D), lambda i: (i, 0))],
        out_specs=pl.BlockSpec((2, D), lambda i: (0, 0)),
        out_shape=jax.ShapeDtypeStruct((2, D), _f32),
    )(x)

    xp = jnp.pad(x, ((0, NP - N), (0, 0)))
    nb = NP // 8

    full_w = pl.BlockSpec((D, D), lambda i: (0, 0))
    node_blk = pl.BlockSpec((nb, D), lambda i: (i, 0))
    part_blk0 = pl.BlockSpec((1, nb, D), lambda i: (0, i, 0))
    part_blk1 = pl.BlockSpec((1, nb, D), lambda i: (1, i, 0))

    us, un = pl.pallas_call(
        _norm_mm_body,
        grid=(8,),
        in_specs=[pl.BlockSpec((2, D), lambda i: (0, 0)), node_blk, full_w, full_w],
        out_specs=[node_blk, node_blk],
        out_shape=[jax.ShapeDtypeStruct((NP, D), _f32)] * 2,
    )(stats, xp, Wself0, Wnbr0)

    e0 = edge_ids[0].astype(jnp.int32)
    e1 = edge_ids[1].astype(jnp.int32)
    src = jnp.concatenate([e0, e1, jnp.zeros((E2P - 2 * E,), jnp.int32)])
    dst = jnp.concatenate([e1, e0, jnp.full((E2P - 2 * E,), NP - 1, jnp.int32)])
    sd = jnp.stack([src, dst])
    zeros_np = jnp.zeros((NP, D), _f32)

    parts0 = _segsum(un, sd, zeros_np)

    vs, vn = pl.pallas_call(
        _layer_body,
        grid=(8,),
        in_specs=[node_blk, part_blk0, part_blk1, full_w, full_w],
        out_specs=[node_blk, node_blk],
        out_shape=[jax.ShapeDtypeStruct((NP, D), _f32)] * 2,
    )(us, parts0, parts0, Wself1, Wnbr1)

    parts1 = _segsum(vn, sd, zeros_np)

    tbl_blk = pl.BlockSpec((nb, 2 * D), lambda i: (i, 0))
    qs_t, qd_t = pl.pallas_call(
        _tables_body,
        grid=(8,),
        in_specs=[node_blk, part_blk0, part_blk1, full_w, full_w],
        out_specs=[tbl_blk, tbl_blk],
        out_shape=[jax.ShapeDtypeStruct((NP, 2 * D), _f32)] * 2,
    )(vs, parts1, parts1, We1[:D], We1[D:2 * D])

    epad = jnp.zeros((EP - E,), jnp.int32)
    sidx = jnp.concatenate([e0, epad])
    didx = jnp.concatenate([e1, epad])
    angp = jnp.concatenate([edge_angles.astype(_f32), jnp.zeros((EP - E,), _f32)])
    wcc = jnp.stack([We1[2 * D], We2[:, 0], We2[:, 1]])

    mu_raw, sr_raw, dotv = _edge_head(qs_t, qd_t, sidx, didx, angp, wcc)

    eps = jax.random.normal(jax.random.key(1), (E, 1), _f32)
    epsp = jnp.concatenate([eps[:, 0], jnp.zeros((EP - E,), _f32)]).reshape(EP // D, D)
    gtp = jnp.concatenate([gt_edge_weights.astype(_f32),
                           jnp.zeros((EP - E,), _f32)]).reshape(EP // D, D)

    eb = EP // D // 8
    edge_blk = pl.BlockSpec((eb, D), lambda i: (i, 0))
    mu_t, std_t, act_t, side = pl.pallas_call(
        _epilogue_body,
        grid=(8,),
        in_specs=[edge_blk] * 5,
        out_specs=[edge_blk, edge_blk, edge_blk,
                   pl.BlockSpec(memory_space=pltpu.SMEM)],
        out_shape=[jax.ShapeDtypeStruct((EP // D, D), _f32)] * 3
                  + [jax.ShapeDtypeStruct((1, 1), _f32)],
    )(mu_raw.reshape(EP // D, D), sr_raw.reshape(EP // D, D),
      dotv.reshape(EP // D, D), epsp, gtp)

    mu = mu_t.reshape(EP)[:E].reshape(E, 1)
    std = std_t.reshape(EP)[:E].reshape(E, 1)
    actions = act_t.reshape(EP)[:E].reshape(E, 1)
    return (mu, std, actions, side[0, 0])


# speedup vs baseline: 1.1421x; 1.1421x over previous
import functools

import jax
import jax.numpy as jnp
from jax import lax
from jax.experimental import pallas as pl
from jax.experimental.pallas import tpu as pltpu
from jax.experimental.pallas import tpu_sc as plsc

N = 10000
D = 128
E = 160000
NP = 10240
EP = 163840
E2P = 327680
NC = 2
NS = 16
ROWS_PER_SUB = NP // NS
SEG_CHUNK = 128
SEG_ITERS = E2P // (NC * NS * SEG_CHUNK)
EDGE_CHUNK = 80
EDGE_ITERS = EP // (NC * NS * EDGE_CHUNK)

SEG_M0 = 110
SEG_M1 = 50
EDGE_M0 = 76
EDGE_M1 = 52

_f32 = jnp.float32


def _stats_body(x_ref, o_ref):
    i = pl.program_id(0)

    @pl.when(i == 0)
    def _init():
        o_ref[...] = jnp.zeros_like(o_ref)

    xb = x_ref[...]
    o_ref[0:1, :] += jnp.sum(xb, axis=0, keepdims=True)
    o_ref[1:2, :] += jnp.sum(xb * xb, axis=0, keepdims=True)


def _norm_mm_body(stats_ref, x_ref, ws_ref, wn_ref, us_ref, un_ref):
    tot = float(N * D)
    m = jnp.sum(stats_ref[0, :]) / tot
    ex2 = jnp.sum(stats_ref[1, :]) / tot
    s = jnp.sqrt(jnp.maximum(ex2 - m * m, 0.0))
    xn = (x_ref[...] - m) * (1.0 / (s + 1e-6))
    us_ref[...] = jnp.dot(xn, ws_ref[...], preferred_element_type=_f32)
    un_ref[...] = jnp.dot(xn, wn_ref[...], preferred_element_type=_f32)


def _layer_body(u_ref, p0_ref, p1_ref, ws_ref, wn_ref, vs_ref, vn_ref):
    h = jnp.maximum(u_ref[...] + p0_ref[0] + p1_ref[0], 0.0)
    vs_ref[...] = jnp.dot(h, ws_ref[...], preferred_element_type=_f32)
    vn_ref[...] = jnp.dot(h, wn_ref[...], preferred_element_type=_f32)


def _tables_body(v_ref, p0_ref, p1_ref, wa_ref, wb_ref, qs_ref, qd_ref):
    h = jnp.maximum(v_ref[...] + p0_ref[0] + p1_ref[0], 0.0)
    ha = jnp.dot(h, wa_ref[...], preferred_element_type=_f32)
    hb = jnp.dot(h, wb_ref[...], preferred_element_type=_f32)
    qs_ref[...] = jnp.concatenate([h, ha], axis=1)
    qd_ref[...] = jnp.concatenate([h, hb], axis=1)


def _epilogue_body(mr_ref, sr_ref, dt_ref, eps_ref, gt_ref,
                   mu_ref, std_ref, act_ref, side_ref):
    i = pl.program_id(0)

    @pl.when(i == 0)
    def _init():
        side_ref[0, 0] = 0.0

    mu = -0.5 + 0.5 * (jnp.tanh(mr_ref[...]) + 1.0)
    std = 0.1 + 0.45 * (jnp.tanh(sr_ref[...]) + 1.0)
    act = jax.nn.sigmoid(mu + std * eps_ref[...])
    mu_ref[...] = mu
    std_ref[...] = std
    act_ref[...] = act

    rows = lax.broadcasted_iota(jnp.int32, mr_ref.shape, 0) + i * mr_ref.shape[0]
    mask = rows < (E // D)
    sp = jax.nn.sigmoid(dt_ref[...])
    side_ref[0, 0] += jnp.sum(jnp.where(mask, (sp - gt_ref[...]) ** 2, 0.0))

    @pl.when(i == pl.num_programs(0) - 1)
    def _fin():
        side_ref[0, 0] = side_ref[0, 0] / float(E)


_sc_mesh = plsc.VectorSubcoreMesh(core_axis_name="c", subcore_axis_name="s")
_sc_params = pltpu.CompilerParams(needs_layout_passes=False)


@functools.partial(
    pl.kernel,
    out_type=jax.ShapeDtypeStruct((NC, NP, D), _f32),
    mesh=_sc_mesh,
    compiler_params=_sc_params,
    scratch_types=[
        pltpu.VMEM_SHARED((NP, D), _f32),
        pltpu.VMEM((2, 2, SEG_CHUNK), jnp.int32),
        pltpu.VMEM((2, SEG_CHUNK, D), _f32),
        pltpu.SemaphoreType.DMA,
        pltpu.SemaphoreType.DMA,
        pltpu.SemaphoreType.DMA,
        pltpu.SemaphoreType.DMA,
    ],
)
def _segsum(table, sd, zeros, out, acc, sdb, rows, si0, si1, sg0, sg1):
    c = lax.axis_index("c")
    s = lax.axis_index("s")
    pltpu.sync_copy(zeros.at[pl.ds(s * ROWS_PER_SUB, ROWS_PER_SUB)],
                    acc.at[pl.ds(s * ROWS_PER_SUB, ROWS_PER_SUB)])
    plsc.subcore_barrier()

    K = SEG_CHUNK
    nchunk = jnp.where(c == 0, SEG_M0, SEG_M1)
    base = jnp.where(c == 0, s * SEG_M0, NS * SEG_M0 + s * SEG_M1) * K

    def issue_idx(buf, off, sem):
        pltpu.async_copy(sd.at[:, pl.ds(off, K)], sdb.at[buf], sem)

    def wait_idx(buf, sem):
        pltpu.make_async_copy(sd.at[:, pl.ds(0, K)], sdb.at[buf], sem).wait()

    def issue_gather(buf, sem):
        pltpu.async_copy(table.at[sdb.at[buf, 0]], rows.at[buf], sem)

    def wait_gather(buf, sem):
        pltpu.make_async_copy(table.at[pl.ds(0, K)], rows.at[buf], sem).wait()

    def scatter(buf):
        pltpu.sync_copy(rows.at[buf], acc.at[sdb.at[buf, 1]], add=True)

    issue_idx(0, base, si0)
    wait_idx(0, si0)
    issue_gather(0, sg0)
    issue_idx(1, base + K, si1)

    npair = nchunk // 2

    def body(j, carry):
        more = j < npair - 1
        wait_gather(0, sg0)
        wait_idx(1, si1)
        issue_gather(1, sg1)
        scatter(0)

        @pl.when(more)
        def _():
            issue_idx(0, base + (2 * j + 2) * K, si0)

        wait_gather(1, sg1)

        @pl.when(more)
        def _():
            wait_idx(0, si0)
            issue_gather(0, sg0)

        scatter(1)

        @pl.when(more)
        def _():
            issue_idx(1, base + (2 * j + 3) * K, si1)

        return carry

    lax.fori_loop(0, npair, body, 0)
    plsc.subcore_barrier()

    for j in range(ROWS_PER_SUB // K):
        r0 = s * ROWS_PER_SUB + j * K
        pltpu.sync_copy(acc.at[pl.ds(r0, K)], rows.at[0])
        pltpu.sync_copy(rows.at[0], out.at[c, pl.ds(r0, K)])


@functools.partial(
    pl.kernel,
    out_type=(jax.ShapeDtypeStruct((EP,), _f32),
              jax.ShapeDtypeStruct((EP,), _f32),
              jax.ShapeDtypeStruct((EP,), _f32)),
    mesh=_sc_mesh,
    compiler_params=_sc_params,
    scratch_types=[
        pltpu.VMEM((2, EDGE_CHUNK, 2 * D), _f32),
        pltpu.VMEM((2, EDGE_CHUNK, 2 * D), _f32),
        pltpu.VMEM((2, EDGE_CHUNK), jnp.int32),
        pltpu.VMEM((2, EDGE_CHUNK), jnp.int32),
        pltpu.VMEM((2, EDGE_CHUNK), _f32),
        pltpu.VMEM((EDGE_CHUNK,), _f32),
        pltpu.VMEM((EDGE_CHUNK,), _f32),
        pltpu.VMEM((EDGE_CHUNK,), _f32),
        pltpu.VMEM((3, D), _f32),
        pltpu.SemaphoreType.DMA,
        pltpu.SemaphoreType.DMA,
        pltpu.SemaphoreType.DMA,
        pltpu.SemaphoreType.DMA,
    ],
)
def _edge_head(qs_t, qd_t, sidx, didx, ang, wcc, mu_o, sr_o, dt_o,
               qs_b, qd_b, sib, dib, anb, mub, srb, dtb, wcc_b,
               si0, si1, sq0, sq1):
    c = lax.axis_index("c")
    s = lax.axis_index("s")
    pltpu.sync_copy(wcc, wcc_b)
    lanes = lax.iota(jnp.int32, 16)
    wv = [wcc_b[0, pl.ds(fb * 16, 16)] for fb in range(8)]
    c0v = [wcc_b[1, pl.ds(fb * 16, 16)] for fb in range(8)]
    c1v = [wcc_b[2, pl.ds(fb * 16, 16)] for fb in range(8)]

    K = EDGE_CHUNK
    nchunk = jnp.where(c == 0, EDGE_M0, EDGE_M1)
    base = jnp.where(c == 0, s * EDGE_M0, NS * EDGE_M0 + s * EDGE_M1) * K

    def issue_idx(buf, off, sem):
        pltpu.async_copy(sidx.at[pl.ds(off, K)], sib.at[buf], sem)
        pltpu.async_copy(didx.at[pl.ds(off, K)], dib.at[buf], sem)
        pltpu.async_copy(ang.at[pl.ds(off, K)], anb.at[buf], sem)

    def wait_idx(buf, sem):
        pltpu.make_async_copy(sidx.at[pl.ds(0, K)], sib.at[buf], sem).wait()
        pltpu.make_async_copy(sidx.at[pl.ds(0, K)], dib.at[buf], sem).wait()
        pltpu.make_async_copy(ang.at[pl.ds(0, K)], anb.at[buf], sem).wait()

    def issue_gathers(buf, sem):
        pltpu.async_copy(qs_t.at[sib.at[buf]], qs_b.at[buf], sem)
        pltpu.async_copy(qd_t.at[dib.at[buf]], qd_b.at[buf], sem)

    def wait_gathers(buf, sem):
        pltpu.make_async_copy(qs_t.at[pl.ds(0, K)], qs_b.at[buf], sem).wait()
        pltpu.make_async_copy(qs_t.at[pl.ds(0, K)], qd_b.at[buf], sem).wait()

    def compute(buf, off):
        def group(g, carry2):
            g16 = g * 16
            ang_v = anb[buf, pl.ds(g16, 16)]
            mu_vec = jnp.zeros((16,), _f32)
            sr_vec = jnp.zeros((16,), _f32)
            dt_vec = jnp.zeros((16,), _f32)
            for lane in range(16):
                e = g16 + lane
                a_sc = ang_v[lane]
                mu_acc = jnp.zeros((16,), _f32)
                sr_acc = jnp.zeros((16,), _f32)
                dt_acc = jnp.zeros((16,), _f32)
                for fb in range(8):
                    hs = qs_b[buf, e, pl.ds(fb * 16, 16)]
                    hd = qd_b[buf, e, pl.ds(fb * 16, 16)]
                    ha = qs_b[buf, e, pl.ds(D + fb * 16, 16)]
                    hb = qd_b[buf, e, pl.ds(D + fb * 16, 16)]
                    rr = jnp.maximum(ha + hb + a_sc * wv[fb], 0.0)
                    mu_acc = mu_acc + rr * c0v[fb]
                    sr_acc = sr_acc + rr * c1v[fb]
                    dt_acc = dt_acc + hs * hd
                lm = lanes == lane
                mu_vec = jnp.where(lm, jnp.sum(mu_acc), mu_vec)
                sr_vec = jnp.where(lm, jnp.sum(sr_acc), sr_vec)
                dt_vec = jnp.where(lm, jnp.sum(dt_acc), dt_vec)
            mub[pl.ds(g16, 16)] = mu_vec
            srb[pl.ds(g16, 16)] = sr_vec
            dtb[pl.ds(g16, 16)] = dt_vec
            return carry2

        lax.fori_loop(0, K // 16, group, 0)
        pltpu.sync_copy(mub, mu_o.at[pl.ds(off, K)])
        pltpu.sync_copy(srb, sr_o.at[pl.ds(off, K)])
        pltpu.sync_copy(dtb, dt_o.at[pl.ds(off, K)])

    issue_idx(0, base, si0)
    wait_idx(0, si0)
    issue_gathers(0, sq0)
    issue_idx(1, base + K, si1)

    npair = nchunk // 2

    def body(j, carry):
        more = j < npair - 1
        off0 = base + (2 * j) * K
        wait_gathers(0, sq0)
        wait_idx(1, si1)
        issue_gathers(1, sq1)
        compute(0, off0)

        @pl.when(more)
        def _():
            issue_idx(0, base + (2 * j + 2) * K, si0)

        wait_gathers(1, sq1)

        @pl.when(more)
        def _():
            wait_idx(0, si0)
            issue_gathers(0, sq0)

        compute(1, off0 + K)

        @pl.when(more)
        def _():
            issue_idx(1, base + (2 * j + 3) * K, si1)

        return carry

    lax.fori_loop(0, npair, body, 0)


def kernel(node_embeddings, sp_feat, edge_ids, edge_angles, gt_edge_weights,
           Wself0, Wnbr0, Wself1, Wnbr1, We1, We2):
    x = jnp.concatenate([node_embeddings, sp_feat], axis=1).astype(_f32)

    stats = pl.pallas_call(
        _stats_body,
        grid=(10,),
        in_specs=[pl.BlockSpec((N // 10, D), lambda i: (i, 0))],
        out_specs=pl.BlockSpec((2, D), lambda i: (0, 0)),
        out_shape=jax.ShapeDtypeStruct((2, D), _f32),
    )(x)

    xp = jnp.pad(x, ((0, NP - N), (0, 0)))
    nb = NP // 8

    full_w = pl.BlockSpec((D, D), lambda i: (0, 0))
    node_blk = pl.BlockSpec((nb, D), lambda i: (i, 0))
    part_blk0 = pl.BlockSpec((1, nb, D), lambda i: (0, i, 0))
    part_blk1 = pl.BlockSpec((1, nb, D), lambda i: (1, i, 0))

    us, un = pl.pallas_call(
        _norm_mm_body,
        grid=(8,),
        in_specs=[pl.BlockSpec((2, D), lambda i: (0, 0)), node_blk, full_w, full_w],
        out_specs=[node_blk, node_blk],
        out_shape=[jax.ShapeDtypeStruct((NP, D), _f32)] * 2,
    )(stats, xp, Wself0, Wnbr0)

    e0 = edge_ids[0].astype(jnp.int32)
    e1 = edge_ids[1].astype(jnp.int32)
    src = jnp.concatenate([e0, e1, jnp.zeros((E2P - 2 * E,), jnp.int32)])
    dst = jnp.concatenate([e1, e0, jnp.full((E2P - 2 * E,), NP - 1, jnp.int32)])
    sd = jnp.stack([src, dst])
    zeros_np = jnp.zeros((NP, D), _f32)

    parts0 = _segsum(un, sd, zeros_np)

    vs, vn = pl.pallas_call(
        _layer_body,
        grid=(8,),
        in_specs=[node_blk, part_blk0, part_blk1, full_w, full_w],
        out_specs=[node_blk, node_blk],
        out_shape=[jax.ShapeDtypeStruct((NP, D), _f32)] * 2,
    )(us, parts0, parts0, Wself1, Wnbr1)

    parts1 = _segsum(vn, sd, zeros_np)

    tbl_blk = pl.BlockSpec((nb, 2 * D), lambda i: (i, 0))
    qs_t, qd_t = pl.pallas_call(
        _tables_body,
        grid=(8,),
        in_specs=[node_blk, part_blk0, part_blk1, full_w, full_w],
        out_specs=[tbl_blk, tbl_blk],
        out_shape=[jax.ShapeDtypeStruct((NP, 2 * D), _f32)] * 2,
    )(vs, parts1, parts1, We1[:D], We1[D:2 * D])

    epad = jnp.zeros((EP - E,), jnp.int32)
    sidx = jnp.concatenate([e0, epad])
    didx = jnp.concatenate([e1, epad])
    angp = jnp.concatenate([edge_angles.astype(_f32), jnp.zeros((EP - E,), _f32)])
    wcc = jnp.stack([We1[2 * D], We2[:, 0], We2[:, 1]])

    mu_raw, sr_raw, dotv = _edge_head(qs_t, qd_t, sidx, didx, angp, wcc)

    eps = jax.random.normal(jax.random.key(1), (E, 1), _f32)
    epsp = jnp.concatenate([eps[:, 0], jnp.zeros((EP - E,), _f32)]).reshape(EP // D, D)
    gtp = jnp.concatenate([gt_edge_weights.astype(_f32),
                           jnp.zeros((EP - E,), _f32)]).reshape(EP // D, D)

    eb = EP // D // 8
    edge_blk = pl.BlockSpec((eb, D), lambda i: (i, 0))
    mu_t, std_t, act_t, side = pl.pallas_call(
        _epilogue_body,
        grid=(8,),
        in_specs=[edge_blk] * 5,
        out_specs=[edge_blk, edge_blk, edge_blk,
                   pl.BlockSpec(memory_space=pltpu.SMEM)],
        out_shape=[jax.ShapeDtypeStruct((EP // D, D), _f32)] * 3
                  + [jax.ShapeDtypeStruct((1, 1), _f32)],
    )(mu_raw.reshape(EP // D, D), sr_raw.reshape(EP // D, D),
      dotv.reshape(EP // D, D), epsp, gtp)

    mu = mu_t.reshape(EP)[:E].reshape(E, 1)
    std = std_t.reshape(EP)[:E].reshape(E, 1)
    actions = act_t.reshape(EP)[:E].reshape(E, 1)
    return (mu, std, actions, side[0, 0])
